# Initial kernel scaffold; baseline (speedup 1.0000x reference)
import jax
import jax.numpy as jnp
from jax.experimental import pallas as pl


def kernel(x, W_enc, b_enc, W_dec, b_dec):
    raise NotImplementedError("write your pallas kernel here")


# speedup vs baseline: 3.9473x; 3.9473x over previous
import functools

import jax
import jax.numpy as jnp
from jax import lax
from jax.experimental import pallas as pl
from jax.experimental.pallas import tpu as pltpu


def _encode_body(x_ref, w_ref, b_ref, out_ref):
    a = lax.dot_general(
        x_ref[...], w_ref[...], (((1,), (1,)), ((), ())),
        preferred_element_type=jnp.float32)
    a = a + b_ref[0]
    a = jnp.maximum(a, 0.0)
    ai = lax.bitcast_convert_type(a, jnp.int32)
    out_ref[0] = jnp.maximum(ai, 0)


def _search_body(ai_ref, out_ref, *, nblk, batch, bj, fdim, k_total):
    def count_ge(t):
        def body(c, acc):
            blk = ai_ref[c]
            return acc + jnp.sum((blk >= t).astype(jnp.int32))
        return lax.fori_loop(0, nblk, body, jnp.int32(0))

    def chunk_flat(c):
        return (lax.broadcasted_iota(jnp.int32, (batch, bj), 0) * fdim
                + lax.broadcasted_iota(jnp.int32, (batch, bj), 1) + c * bj)

    def count_tie_le(t, u):
        def body(c, acc):
            blk = ai_ref[c]
            m = (blk == t) & (chunk_flat(c) <= u)
            return acc + jnp.sum(m.astype(jnp.int32))
        return lax.fori_loop(0, nblk, body, jnp.int32(0))

    def max_body(c, acc):
        return jnp.maximum(acc, jnp.max(ai_ref[c]))
    maxb = lax.fori_loop(0, nblk, max_body, jnp.int32(0))

    def vcond(s):
        lo, hi = s
        return lo < hi

    def vbody(s):
        lo, hi = s
        mid = lo + (hi - lo + 1) // 2
        big = count_ge(mid) >= k_total
        return jnp.where(big, mid, lo), jnp.where(big, hi, mid - 1)

    tval, _ = lax.while_loop(vcond, vbody, (jnp.int32(0), maxb))

    c_gt = count_ge(tval + 1)
    r = k_total - c_gt

    def min_tie():
        bigidx = jnp.int32(batch * fdim)
        def body(c, acc):
            blk = ai_ref[c]
            cand = jnp.where(blk == tval, chunk_flat(c), bigidx)
            return jnp.minimum(acc, jnp.min(cand))
        return lax.fori_loop(0, nblk, body, bigidx)

    def bisect_tie():
        def icond(s):
            lo, hi = s
            return lo < hi
        def ibody(s):
            lo, hi = s
            mid = lo + (hi - lo) // 2
            enough = count_tie_le(tval, mid) >= r
            return jnp.where(enough, lo, mid + 1), jnp.where(enough, mid, hi)
        lo, _ = lax.while_loop(
            icond, ibody, (jnp.int32(0), jnp.int32(batch * fdim - 1)))
        return lo

    u = lax.cond(r == 1, min_tie, bisect_tie)
    out_ref[0] = tval
    out_ref[1] = u


def _decode_body(tpar_ref, ai_ref, w_ref, out_ref, *, batch, bj, fdim):
    j = pl.program_id(0)
    tval = tpar_ref[0]
    u = tpar_ref[1]
    ai = ai_ref[0]
    flat = (lax.broadcasted_iota(jnp.int32, (batch, bj), 0) * fdim
            + lax.broadcasted_iota(jnp.int32, (batch, bj), 1) + j * bj)
    sel = (ai > tval) | ((ai == tval) & (flat <= u))
    vals = lax.bitcast_convert_type(ai, jnp.float32)
    f = jnp.where(sel, vals, 0.0)
    contrib = lax.dot_general(
        f, w_ref[...], (((1,), (1,)), ((), ())),
        preferred_element_type=jnp.float32)

    @pl.when(j == 0)
    def _():
        out_ref[...] = contrib

    @pl.when(j > 0)
    def _():
        out_ref[...] += contrib


def _run(x_eff, W_enc, b_enc, W_dec, k_total, bj, interpret=False):
    batch, adim = x_eff.shape
    fdim = W_enc.shape[0]
    nblk = fdim // bj

    ai = pl.pallas_call(
        _encode_body,
        grid=(nblk,),
        in_specs=[
            pl.BlockSpec((batch, adim), lambda i: (0, 0)),
            pl.BlockSpec((bj, adim), lambda i: (i, 0)),
            pl.BlockSpec((1, 1, bj), lambda i: (i, 0, 0)),
        ],
        out_specs=pl.BlockSpec((1, batch, bj), lambda i: (i, 0, 0)),
        out_shape=jax.ShapeDtypeStruct((nblk, batch, bj), jnp.int32),
        interpret=interpret,
    )(x_eff, W_enc, b_enc.reshape(nblk, 1, bj))

    tpar = pl.pallas_call(
        functools.partial(_search_body, nblk=nblk, batch=batch, bj=bj,
                          fdim=fdim, k_total=k_total),
        in_specs=[pl.BlockSpec((nblk, batch, bj), lambda: (0, 0, 0))],
        out_specs=pl.BlockSpec(memory_space=pltpu.SMEM),
        out_shape=jax.ShapeDtypeStruct((2,), jnp.int32),
        interpret=interpret,
    )(ai)

    y = pl.pallas_call(
        functools.partial(_decode_body, batch=batch, bj=bj, fdim=fdim),
        grid=(nblk,),
        in_specs=[
            pl.BlockSpec(memory_space=pltpu.SMEM),
            pl.BlockSpec((1, batch, bj), lambda j: (j, 0, 0)),
            pl.BlockSpec((adim, bj), lambda j: (0, j)),
        ],
        out_specs=pl.BlockSpec((batch, adim), lambda j: (0, 0)),
        out_shape=jax.ShapeDtypeStruct((batch, adim), jnp.float32),
        compiler_params=pltpu.CompilerParams(
            dimension_semantics=("arbitrary",)),
        interpret=interpret,
    )(tpar, ai, W_dec)
    return y


def kernel(x, W_enc, b_enc, W_dec, b_dec):
    x_eff = x - b_dec[None, :]
    batch = x.shape[0]
    y = _run(x_eff, W_enc, b_enc, W_dec, k_total=64 * batch, bj=1024)
    return y + b_dec[None, :]


# speedup vs baseline: 6.5416x; 1.6572x over previous
import functools

import jax
import jax.numpy as jnp
from jax import lax
from jax.experimental import pallas as pl
from jax.experimental.pallas import tpu as pltpu


def _encode_body(x_ref, w_ref, b_ref, out_ref):
    a = lax.dot_general(
        x_ref[...], w_ref[...], (((1,), (1,)), ((), ())),
        preferred_element_type=jnp.float32)
    a = a + b_ref[0]
    a = jnp.maximum(a, 0.0)
    ai = lax.bitcast_convert_type(a, jnp.int32)
    out_ref[0] = jnp.maximum(ai, 0)


def _search_body(ai_ref, out_ref, *, nblk, batch, bj, fdim, k_total):
    def fold_add(m):
        s = m[0:8]
        for i in range(1, batch // 8):
            s = s + m[8 * i:8 * (i + 1)]
        t = s[:, 0:128]
        for i in range(1, bj // 128):
            t = t + s[:, 128 * i:128 * (i + 1)]
        return t

    def fold_with(m, op):
        s = m[0:8]
        for i in range(1, batch // 8):
            s = op(s, m[8 * i:8 * (i + 1)])
        t = s[:, 0:128]
        for i in range(1, bj // 128):
            t = op(t, s[:, 128 * i:128 * (i + 1)])
        return t

    def count_ge(t):
        def body(c, acc):
            blk = ai_ref[c]
            return acc + fold_add((blk >= t).astype(jnp.int32))
        acc = lax.fori_loop(0, nblk, body, jnp.zeros((8, 128), jnp.int32))
        return jnp.sum(acc)

    def chunk_flat(c):
        return (lax.broadcasted_iota(jnp.int32, (batch, bj), 0) * fdim
                + lax.broadcasted_iota(jnp.int32, (batch, bj), 1) + c * bj)

    def count_tie_le(t, u):
        def body(c, acc):
            blk = ai_ref[c]
            m = (blk == t) & (chunk_flat(c) <= u)
            return acc + fold_add(m.astype(jnp.int32))
        acc = lax.fori_loop(0, nblk, body, jnp.zeros((8, 128), jnp.int32))
        return jnp.sum(acc)

    def max_body(c, acc):
        return jnp.maximum(acc, fold_with(ai_ref[c], jnp.maximum))
    maxb = jnp.max(lax.fori_loop(
        0, nblk, max_body, jnp.full((8, 128), jnp.int32(0))))

    def vcond(s):
        lo, hi = s
        return lo < hi

    def vbody(s):
        lo, hi = s
        mid = lo + (hi - lo + 1) // 2
        big = count_ge(mid) >= k_total
        return jnp.where(big, mid, lo), jnp.where(big, hi, mid - 1)

    tval, _ = lax.while_loop(vcond, vbody, (jnp.int32(0), maxb))

    c_gt = count_ge(tval + 1)
    r = k_total - c_gt

    def min_tie():
        bigidx = jnp.int32(batch * fdim)
        def body(c, acc):
            blk = ai_ref[c]
            cand = jnp.where(blk == tval, chunk_flat(c), bigidx)
            return jnp.minimum(acc, fold_with(cand, jnp.minimum))
        acc = lax.fori_loop(
            0, nblk, body, jnp.full((8, 128), bigidx))
        return jnp.min(acc)

    def bisect_tie():
        def icond(s):
            lo, hi = s
            return lo < hi
        def ibody(s):
            lo, hi = s
            mid = lo + (hi - lo) // 2
            enough = count_tie_le(tval, mid) >= r
            return jnp.where(enough, lo, mid + 1), jnp.where(enough, mid, hi)
        lo, _ = lax.while_loop(
            icond, ibody, (jnp.int32(0), jnp.int32(batch * fdim - 1)))
        return lo

    u = lax.cond(r == 1, min_tie, bisect_tie)
    out_ref[0] = tval
    out_ref[1] = u


def _decode_body(tpar_ref, ai_ref, w_ref, out_ref, *, batch, bj, fdim):
    j = pl.program_id(0)
    tval = tpar_ref[0]
    u = tpar_ref[1]
    ai = ai_ref[0]
    flat = (lax.broadcasted_iota(jnp.int32, (batch, bj), 0) * fdim
            + lax.broadcasted_iota(jnp.int32, (batch, bj), 1) + j * bj)
    sel = (ai > tval) | ((ai == tval) & (flat <= u))
    vals = lax.bitcast_convert_type(ai, jnp.float32)
    f = jnp.where(sel, vals, 0.0)
    contrib = lax.dot_general(
        f, w_ref[...], (((1,), (1,)), ((), ())),
        preferred_element_type=jnp.float32)

    @pl.when(j == 0)
    def _():
        out_ref[...] = contrib

    @pl.when(j > 0)
    def _():
        out_ref[...] += contrib


def _run(x_eff, W_enc, b_enc, W_dec, k_total, bj, interpret=False):
    batch, adim = x_eff.shape
    fdim = W_enc.shape[0]
    nblk = fdim // bj

    ai = pl.pallas_call(
        _encode_body,
        grid=(nblk,),
        in_specs=[
            pl.BlockSpec((batch, adim), lambda i: (0, 0)),
            pl.BlockSpec((bj, adim), lambda i: (i, 0)),
            pl.BlockSpec((1, 1, bj), lambda i: (i, 0, 0)),
        ],
        out_specs=pl.BlockSpec((1, batch, bj), lambda i: (i, 0, 0)),
        out_shape=jax.ShapeDtypeStruct((nblk, batch, bj), jnp.int32),
        interpret=interpret,
    )(x_eff, W_enc, b_enc.reshape(nblk, 1, bj))

    tpar = pl.pallas_call(
        functools.partial(_search_body, nblk=nblk, batch=batch, bj=bj,
                          fdim=fdim, k_total=k_total),
        in_specs=[pl.BlockSpec((nblk, batch, bj), lambda: (0, 0, 0))],
        out_specs=pl.BlockSpec(memory_space=pltpu.SMEM),
        out_shape=jax.ShapeDtypeStruct((2,), jnp.int32),
        interpret=interpret,
    )(ai)

    y = pl.pallas_call(
        functools.partial(_decode_body, batch=batch, bj=bj, fdim=fdim),
        grid=(nblk,),
        in_specs=[
            pl.BlockSpec(memory_space=pltpu.SMEM),
            pl.BlockSpec((1, batch, bj), lambda j: (j, 0, 0)),
            pl.BlockSpec((adim, bj), lambda j: (0, j)),
        ],
        out_specs=pl.BlockSpec((batch, adim), lambda j: (0, 0)),
        out_shape=jax.ShapeDtypeStruct((batch, adim), jnp.float32),
        compiler_params=pltpu.CompilerParams(
            dimension_semantics=("arbitrary",)),
        interpret=interpret,
    )(tpar, ai, W_dec)
    return y


def kernel(x, W_enc, b_enc, W_dec, b_dec):
    x_eff = x - b_dec[None, :]
    batch = x.shape[0]
    y = _run(x_eff, W_enc, b_enc, W_dec, k_total=64 * batch, bj=1024)
    return y + b_dec[None, :]


# speedup vs baseline: 6.6810x; 1.0213x over previous
import functools

import jax
import jax.numpy as jnp
from jax import lax
from jax.experimental import pallas as pl
from jax.experimental.pallas import tpu as pltpu


def _search_body(ai_ref, out_ref, *, nblk, batch, bj, fdim, k_total):
    def fold_add(m):
        s = m[0:8]
        for i in range(1, batch // 8):
            s = s + m[8 * i:8 * (i + 1)]
        t = s[:, 0:128]
        for i in range(1, bj // 128):
            t = t + s[:, 128 * i:128 * (i + 1)]
        return t

    def fold_with(m, op):
        s = m[0:8]
        for i in range(1, batch // 8):
            s = op(s, m[8 * i:8 * (i + 1)])
        t = s[:, 0:128]
        for i in range(1, bj // 128):
            t = op(t, s[:, 128 * i:128 * (i + 1)])
        return t

    def count_ge(t):
        def body(c, acc):
            blk = ai_ref[c]
            return acc + fold_add((blk >= t).astype(jnp.int32))
        acc = lax.fori_loop(0, nblk, body, jnp.zeros((8, 128), jnp.int32))
        return jnp.sum(acc)

    def chunk_flat(c):
        return (lax.broadcasted_iota(jnp.int32, (batch, bj), 0) * fdim
                + lax.broadcasted_iota(jnp.int32, (batch, bj), 1) + c * bj)

    def count_tie_le(t, u):
        def body(c, acc):
            blk = ai_ref[c]
            m = (blk == t) & (chunk_flat(c) <= u)
            return acc + fold_add(m.astype(jnp.int32))
        acc = lax.fori_loop(0, nblk, body, jnp.zeros((8, 128), jnp.int32))
        return jnp.sum(acc)

    def max_body(c, acc):
        return jnp.maximum(acc, fold_with(ai_ref[c], jnp.maximum))
    maxb = jnp.max(lax.fori_loop(
        0, nblk, max_body, jnp.full((8, 128), jnp.int32(0))))

    def vcond(s):
        lo, hi = s
        return lo < hi

    def vbody(s):
        lo, hi = s
        mid = lo + (hi - lo + 1) // 2
        big = count_ge(mid) >= k_total
        return jnp.where(big, mid, lo), jnp.where(big, hi, mid - 1)

    tval, _ = lax.while_loop(vcond, vbody, (jnp.int32(0), maxb))

    c_gt = count_ge(tval + 1)
    r = k_total - c_gt

    def min_tie():
        bigidx = jnp.int32(batch * fdim)
        def body(c, acc):
            blk = ai_ref[c]
            cand = jnp.where(blk == tval, chunk_flat(c), bigidx)
            return jnp.minimum(acc, fold_with(cand, jnp.minimum))
        acc = lax.fori_loop(
            0, nblk, body, jnp.full((8, 128), bigidx))
        return jnp.min(acc)

    def bisect_tie():
        def icond(s):
            lo, hi = s
            return lo < hi
        def ibody(s):
            lo, hi = s
            mid = lo + (hi - lo) // 2
            enough = count_tie_le(tval, mid) >= r
            return jnp.where(enough, lo, mid + 1), jnp.where(enough, mid, hi)
        lo, _ = lax.while_loop(
            icond, ibody, (jnp.int32(0), jnp.int32(batch * fdim - 1)))
        return lo

    u = lax.cond(r == 1, min_tie, bisect_tie)
    out_ref[0] = tval
    out_ref[1] = u


def _fused_body(x_ref, w_enc_ref, b_ref, w_dec_ref, out_ref, ai_scr, tpar,
                *, nblk, batch, bj, fdim, k_total):
    g = pl.program_id(0)

    @pl.when(g < nblk)
    def _encode():
        a = lax.dot_general(
            x_ref[...], w_enc_ref[...], (((1,), (1,)), ((), ())),
            preferred_element_type=jnp.float32)
        a = a + b_ref[0]
        a = jnp.maximum(a, 0.0)
        ai = lax.bitcast_convert_type(a, jnp.int32)
        ai_scr[g] = jnp.maximum(ai, 0)

    @pl.when(g == nblk)
    def _search():
        _search_body(ai_scr, tpar, nblk=nblk, batch=batch, bj=bj,
                     fdim=fdim, k_total=k_total)

    @pl.when(g > nblk)
    def _decode():
        j = g - (nblk + 1)
        tval = tpar[0]
        u = tpar[1]
        ai = ai_scr[j]
        flat = (lax.broadcasted_iota(jnp.int32, (batch, bj), 0) * fdim
                + lax.broadcasted_iota(jnp.int32, (batch, bj), 1) + j * bj)
        sel = (ai > tval) | ((ai == tval) & (flat <= u))
        vals = lax.bitcast_convert_type(ai, jnp.float32)
        f = jnp.where(sel, vals, 0.0)
        contrib = lax.dot_general(
            f, w_dec_ref[...], (((1,), (1,)), ((), ())),
            preferred_element_type=jnp.float32)

        @pl.when(j == 0)
        def _():
            out_ref[...] = contrib

        @pl.when(j > 0)
        def _():
            out_ref[...] += contrib


def _run(x_eff, W_enc, b_enc, W_dec, k_total, bj, interpret=False):
    batch, adim = x_eff.shape
    fdim = W_enc.shape[0]
    nblk = fdim // bj

    y = pl.pallas_call(
        functools.partial(_fused_body, nblk=nblk, batch=batch, bj=bj,
                          fdim=fdim, k_total=k_total),
        grid=(2 * nblk + 1,),
        in_specs=[
            pl.BlockSpec((batch, adim), lambda g: (0, 0)),
            pl.BlockSpec((bj, adim), lambda g: (jnp.minimum(g, nblk - 1), 0)),
            pl.BlockSpec((1, 1, bj),
                         lambda g: (jnp.minimum(g, nblk - 1), 0, 0)),
            pl.BlockSpec((adim, bj),
                         lambda g: (0, jnp.clip(g - (nblk + 1), 0, nblk - 1))),
        ],
        out_specs=pl.BlockSpec((batch, adim), lambda g: (0, 0)),
        out_shape=jax.ShapeDtypeStruct((batch, adim), jnp.float32),
        scratch_shapes=[
            pltpu.VMEM((nblk, batch, bj), jnp.int32),
            pltpu.SMEM((2,), jnp.int32),
        ],
        compiler_params=pltpu.CompilerParams(
            dimension_semantics=("arbitrary",)),
        interpret=interpret,
    )(x_eff, W_enc, b_enc.reshape(nblk, 1, bj), W_dec)
    return y


def kernel(x, W_enc, b_enc, W_dec, b_dec):
    x_eff = x - b_dec[None, :]
    batch = x.shape[0]
    y = _run(x_eff, W_enc, b_enc, W_dec, k_total=64 * batch, bj=1024)
    return y + b_dec[None, :]


# speedup vs baseline: 6.8840x; 1.0304x over previous
import functools

import jax
import jax.numpy as jnp
from jax import lax
from jax.experimental import pallas as pl
from jax.experimental.pallas import tpu as pltpu


def _search_body(ai_ref, g12_ref, out_ref, *, nblk, batch, bj, fdim, k_total):
    def fold_add(m):
        s = m[0:8]
        for i in range(1, batch // 8):
            s = s + m[8 * i:8 * (i + 1)]
        t = s[:, 0:128]
        for i in range(1, bj // 128):
            t = t + s[:, 128 * i:128 * (i + 1)]
        return t

    def fold_with(m, op):
        s = m[0:8]
        for i in range(1, batch // 8):
            s = op(s, m[8 * i:8 * (i + 1)])
        t = s[:, 0:128]
        for i in range(1, bj // 128):
            t = op(t, s[:, 128 * i:128 * (i + 1)])
        return t

    def count_ge(t):
        def body(c, acc):
            blk = ai_ref[c]
            return acc + fold_add((blk >= t).astype(jnp.int32))
        acc = lax.fori_loop(0, nblk, body, jnp.zeros((8, 128), jnp.int32))
        return jnp.sum(acc)

    def count_ge12(t):
        def body(c, acc):
            blk = g12_ref[c]
            m = (blk >= t).astype(jnp.int32)
            s = m[0:8] + m[8:16]
            u = s[:, 0:128]
            for i in range(1, bj // 128):
                u = u + s[:, 128 * i:128 * (i + 1)]
            return acc + u
        acc = lax.fori_loop(0, nblk, body, jnp.zeros((8, 128), jnp.int32))
        return jnp.sum(acc)

    def max12():
        def body(c, acc):
            blk = g12_ref[c]
            s = jnp.maximum(blk[0:8], blk[8:16])
            u = s[:, 0:128]
            for i in range(1, bj // 128):
                u = jnp.maximum(u, s[:, 128 * i:128 * (i + 1)])
            return jnp.maximum(acc, u)
        acc = lax.fori_loop(0, nblk, body, jnp.zeros((8, 128), jnp.int32))
        return jnp.max(acc)

    def chunk_flat(c):
        return (lax.broadcasted_iota(jnp.int32, (batch, bj), 0) * fdim
                + lax.broadcasted_iota(jnp.int32, (batch, bj), 1) + c * bj)

    def count_tie_le(t, u):
        def body(c, acc):
            blk = ai_ref[c]
            m = (blk == t) & (chunk_flat(c) <= u)
            return acc + fold_add(m.astype(jnp.int32))
        acc = lax.fori_loop(0, nblk, body, jnp.zeros((8, 128), jnp.int32))
        return jnp.sum(acc)

    maxb = max12()

    def vcond(s):
        lo, hi = s
        return lo < hi

    def vbody12(s):
        lo, hi = s
        mid = lo + (hi - lo + 1) // 2
        big = count_ge12(mid) >= k_total
        return jnp.where(big, mid, lo), jnp.where(big, hi, mid - 1)

    tcand, _ = lax.while_loop(vcond, vbody12, (jnp.int32(0), maxb))

    ca = count_ge(tcand)
    cb = count_ge(tcand + 1)
    ok = (ca >= k_total) & (cb < k_total)

    def fallback():
        def vbody(s):
            lo, hi = s
            mid = lo + (hi - lo + 1) // 2
            big = count_ge(mid) >= k_total
            return jnp.where(big, mid, lo), jnp.where(big, hi, mid - 1)
        t_fb, _ = lax.while_loop(vcond, vbody, (jnp.int32(0), maxb))
        return t_fb, count_ge(t_fb + 1)

    tval, c_gt = lax.cond(ok, lambda: (tcand, cb), fallback)
    r = k_total - c_gt

    def min_tie():
        bigidx = jnp.int32(batch * fdim)
        def body(c, acc):
            blk = ai_ref[c]
            cand = jnp.where(blk == tval, chunk_flat(c), bigidx)
            return jnp.minimum(acc, fold_with(cand, jnp.minimum))
        acc = lax.fori_loop(
            0, nblk, body, jnp.full((8, 128), bigidx))
        return jnp.min(acc)

    def bisect_tie():
        def icond(s):
            lo, hi = s
            return lo < hi
        def ibody(s):
            lo, hi = s
            mid = lo + (hi - lo) // 2
            enough = count_tie_le(tval, mid) >= r
            return jnp.where(enough, lo, mid + 1), jnp.where(enough, mid, hi)
        lo, _ = lax.while_loop(
            icond, ibody, (jnp.int32(0), jnp.int32(batch * fdim - 1)))
        return lo

    u = lax.cond(r == 1, min_tie, bisect_tie)
    out_ref[0] = tval
    out_ref[1] = u


def _fused_body(x_ref, w_enc_ref, b_ref, w_dec_ref, out_ref, ai_scr, g12_scr,
                tpar, *, nblk, batch, bj, fdim, k_total):
    g = pl.program_id(0)

    @pl.when(g < nblk)
    def _encode():
        a = lax.dot_general(
            x_ref[...], w_enc_ref[...], (((1,), (1,)), ((), ())),
            preferred_element_type=jnp.float32)
        a = a + b_ref[0]
        a = jnp.maximum(a, 0.0)
        ai = lax.bitcast_convert_type(a, jnp.int32)
        ai = jnp.maximum(ai, 0)
        ai_scr[g] = ai
        nr = batch // 8
        if nr == 4:
            r0, r1 = ai[0:8], ai[8:16]
            r2, r3 = ai[16:24], ai[24:32]
            s1, i1 = jnp.maximum(r0, r1), jnp.minimum(r0, r1)
            s2, i2 = jnp.maximum(r2, r3), jnp.minimum(r2, r3)
            g1 = jnp.maximum(s1, s2)
            g2 = jnp.maximum(jnp.minimum(s1, s2), jnp.maximum(i1, i2))
        elif nr == 2:
            g1 = jnp.maximum(ai[0:8], ai[8:16])
            g2 = jnp.minimum(ai[0:8], ai[8:16])
        else:
            g1 = ai[0:8]
            g2 = jnp.zeros_like(g1)
        g12_scr[g] = jnp.concatenate([g1, g2], axis=0)

    @pl.when(g == nblk)
    def _search():
        _search_body(ai_scr, g12_scr, tpar, nblk=nblk, batch=batch, bj=bj,
                     fdim=fdim, k_total=k_total)

    @pl.when(g > nblk)
    def _decode():
        j = g - (nblk + 1)
        tval = tpar[0]
        u = tpar[1]
        ai = ai_scr[j]
        flat = (lax.broadcasted_iota(jnp.int32, (batch, bj), 0) * fdim
                + lax.broadcasted_iota(jnp.int32, (batch, bj), 1) + j * bj)
        sel = (ai > tval) | ((ai == tval) & (flat <= u))
        vals = lax.bitcast_convert_type(ai, jnp.float32)
        f = jnp.where(sel, vals, 0.0)
        contrib = lax.dot_general(
            f, w_dec_ref[...], (((1,), (1,)), ((), ())),
            preferred_element_type=jnp.float32)

        @pl.when(j == 0)
        def _():
            out_ref[...] = contrib

        @pl.when(j > 0)
        def _():
            out_ref[...] += contrib


def _run(x_eff, W_enc, b_enc, W_dec, k_total, bj, interpret=False):
    batch, adim = x_eff.shape
    fdim = W_enc.shape[0]
    nblk = fdim // bj

    y = pl.pallas_call(
        functools.partial(_fused_body, nblk=nblk, batch=batch, bj=bj,
                          fdim=fdim, k_total=k_total),
        grid=(2 * nblk + 1,),
        in_specs=[
            pl.BlockSpec((batch, adim), lambda g: (0, 0)),
            pl.BlockSpec((bj, adim), lambda g: (jnp.minimum(g, nblk - 1), 0)),
            pl.BlockSpec((1, 1, bj),
                         lambda g: (jnp.minimum(g, nblk - 1), 0, 0)),
            pl.BlockSpec((adim, bj),
                         lambda g: (0, jnp.clip(g - (nblk + 1), 0, nblk - 1))),
        ],
        out_specs=pl.BlockSpec((batch, adim), lambda g: (0, 0)),
        out_shape=jax.ShapeDtypeStruct((batch, adim), jnp.float32),
        scratch_shapes=[
            pltpu.VMEM((nblk, batch, bj), jnp.int32),
            pltpu.VMEM((nblk, 16, bj), jnp.int32),
            pltpu.SMEM((2,), jnp.int32),
        ],
        compiler_params=pltpu.CompilerParams(
            dimension_semantics=("arbitrary",)),
        interpret=interpret,
    )(x_eff, W_enc, b_enc.reshape(nblk, 1, bj), W_dec)
    return y


def kernel(x, W_enc, b_enc, W_dec, b_dec):
    x_eff = x - b_dec[None, :]
    batch = x.shape[0]
    y = _run(x_eff, W_enc, b_enc, W_dec, k_total=64 * batch, bj=1024)
    return y + b_dec[None, :]


# speedup vs baseline: 7.0983x; 1.0311x over previous
import functools

import jax
import jax.numpy as jnp
from jax import lax
from jax.experimental import pallas as pl
from jax.experimental.pallas import tpu as pltpu


def _search_body(ai_ref, g12_ref, out_ref, *, nblk, batch, bj, fdim, k_total):
    def fold_add(m):
        s = m[0:8]
        for i in range(1, batch // 8):
            s = s + m[8 * i:8 * (i + 1)]
        t = s[:, 0:128]
        for i in range(1, bj // 128):
            t = t + s[:, 128 * i:128 * (i + 1)]
        return t

    def fold_with(m, op):
        s = m[0:8]
        for i in range(1, batch // 8):
            s = op(s, m[8 * i:8 * (i + 1)])
        t = s[:, 0:128]
        for i in range(1, bj // 128):
            t = op(t, s[:, 128 * i:128 * (i + 1)])
        return t

    def count_ge(t):
        def body(c, acc):
            blk = ai_ref[c]
            return acc + fold_add((blk >= t).astype(jnp.int32))
        acc = lax.fori_loop(0, nblk, body, jnp.zeros((8, 128), jnp.int32))
        return jnp.sum(acc)

    wc = bj // 4
    wf = min(wc, 128)

    def count_ge12(t):
        def body(c, acc):
            blk = g12_ref[c]
            m = (blk >= t).astype(jnp.int32)
            s = m[0:8] + m[8:16]
            u = s[:, 0:wf]
            for i in range(1, wc // 128):
                u = u + s[:, 128 * i:128 * (i + 1)]
            return acc + u
        acc = lax.fori_loop(0, nblk, body, jnp.zeros((8, wf), jnp.int32))
        return jnp.sum(acc)

    def max12():
        def body(c, acc):
            blk = g12_ref[c]
            s = jnp.maximum(blk[0:8], blk[8:16])
            u = s[:, 0:wf]
            for i in range(1, wc // 128):
                u = jnp.maximum(u, s[:, 128 * i:128 * (i + 1)])
            return jnp.maximum(acc, u)
        acc = lax.fori_loop(0, nblk, body, jnp.zeros((8, wf), jnp.int32))
        return jnp.max(acc)

    def chunk_flat(c):
        return (lax.broadcasted_iota(jnp.int32, (batch, bj), 0) * fdim
                + lax.broadcasted_iota(jnp.int32, (batch, bj), 1) + c * bj)

    def count_tie_le(t, u):
        def body(c, acc):
            blk = ai_ref[c]
            m = (blk == t) & (chunk_flat(c) <= u)
            return acc + fold_add(m.astype(jnp.int32))
        acc = lax.fori_loop(0, nblk, body, jnp.zeros((8, 128), jnp.int32))
        return jnp.sum(acc)

    maxb = max12()

    def vcond(s):
        lo, hi = s
        return lo < hi

    def vbody12(s):
        lo, hi = s
        mid = lo + (hi - lo + 1) // 2
        big = count_ge12(mid) >= k_total
        return jnp.where(big, mid, lo), jnp.where(big, hi, mid - 1)

    tcand, _ = lax.while_loop(vcond, vbody12, (jnp.int32(0), maxb))

    ca = count_ge(tcand)
    cb = count_ge(tcand + 1)
    ok = (ca >= k_total) & (cb < k_total)

    def fallback():
        def vbody(s):
            lo, hi = s
            mid = lo + (hi - lo + 1) // 2
            big = count_ge(mid) >= k_total
            return jnp.where(big, mid, lo), jnp.where(big, hi, mid - 1)
        t_fb, _ = lax.while_loop(vcond, vbody, (jnp.int32(0), maxb))
        return t_fb, count_ge(t_fb + 1)

    tval, c_gt = lax.cond(ok, lambda: (tcand, cb), fallback)
    r = k_total - c_gt

    def min_tie():
        bigidx = jnp.int32(batch * fdim)
        def body(c, acc):
            blk = ai_ref[c]
            cand = jnp.where(blk == tval, chunk_flat(c), bigidx)
            return jnp.minimum(acc, fold_with(cand, jnp.minimum))
        acc = lax.fori_loop(
            0, nblk, body, jnp.full((8, 128), bigidx))
        return jnp.min(acc)

    def bisect_tie():
        def icond(s):
            lo, hi = s
            return lo < hi
        def ibody(s):
            lo, hi = s
            mid = lo + (hi - lo) // 2
            enough = count_tie_le(tval, mid) >= r
            return jnp.where(enough, lo, mid + 1), jnp.where(enough, mid, hi)
        lo, _ = lax.while_loop(
            icond, ibody, (jnp.int32(0), jnp.int32(batch * fdim - 1)))
        return lo

    u = lax.cond(r == 1, min_tie, bisect_tie)
    out_ref[0] = tval
    out_ref[1] = u


def _fused_body(x_ref, w_enc_ref, b_ref, w_dec_ref, out_ref, ai_scr, g12_scr,
                tpar, *, nblk, batch, bj, fdim, k_total):
    g = pl.program_id(0)

    @pl.when(g < nblk)
    def _encode():
        a = lax.dot_general(
            x_ref[...], w_enc_ref[...], (((1,), (1,)), ((), ())),
            preferred_element_type=jnp.float32)
        a = a + b_ref[0]
        a = jnp.maximum(a, 0.0)
        ai = lax.bitcast_convert_type(a, jnp.int32)
        ai = jnp.maximum(ai, 0)
        ai_scr[g] = ai
        nr = batch // 8
        if nr == 4:
            r0, r1 = ai[0:8], ai[8:16]
            r2, r3 = ai[16:24], ai[24:32]
            s1, i1 = jnp.maximum(r0, r1), jnp.minimum(r0, r1)
            s2, i2 = jnp.maximum(r2, r3), jnp.minimum(r2, r3)
            g1 = jnp.maximum(s1, s2)
            g2 = jnp.maximum(jnp.minimum(s1, s2), jnp.maximum(i1, i2))
        elif nr == 2:
            g1 = jnp.maximum(ai[0:8], ai[8:16])
            g2 = jnp.minimum(ai[0:8], ai[8:16])
        else:
            g1 = ai[0:8]
            g2 = jnp.zeros_like(g1)
        w = bj
        for _ in range(2):
            h = w // 2
            g1a, g1b = g1[:, :h], g1[:, h:]
            g2a, g2b = g2[:, :h], g2[:, h:]
            g1, g2 = (jnp.maximum(g1a, g1b),
                      jnp.maximum(jnp.minimum(g1a, g1b),
                                  jnp.maximum(g2a, g2b)))
            w = h
        g12_scr[g] = jnp.concatenate([g1, g2], axis=0)

    @pl.when(g == nblk)
    def _search():
        _search_body(ai_scr, g12_scr, tpar, nblk=nblk, batch=batch, bj=bj,
                     fdim=fdim, k_total=k_total)

    @pl.when(g > nblk)
    def _decode():
        j = g - (nblk + 1)
        tval = tpar[0]
        u = tpar[1]
        ai = ai_scr[j]
        flat = (lax.broadcasted_iota(jnp.int32, (batch, bj), 0) * fdim
                + lax.broadcasted_iota(jnp.int32, (batch, bj), 1) + j * bj)
        sel = (ai > tval) | ((ai == tval) & (flat <= u))
        vals = lax.bitcast_convert_type(ai, jnp.float32)
        f = jnp.where(sel, vals, 0.0)
        contrib = lax.dot_general(
            f, w_dec_ref[...], (((1,), (1,)), ((), ())),
            preferred_element_type=jnp.float32)

        @pl.when(j == 0)
        def _():
            out_ref[...] = contrib

        @pl.when(j > 0)
        def _():
            out_ref[...] += contrib


def _run(x_eff, W_enc, b_enc, W_dec, k_total, bj, interpret=False):
    batch, adim = x_eff.shape
    fdim = W_enc.shape[0]
    nblk = fdim // bj

    y = pl.pallas_call(
        functools.partial(_fused_body, nblk=nblk, batch=batch, bj=bj,
                          fdim=fdim, k_total=k_total),
        grid=(2 * nblk + 1,),
        in_specs=[
            pl.BlockSpec((batch, adim), lambda g: (0, 0)),
            pl.BlockSpec((bj, adim), lambda g: (jnp.minimum(g, nblk - 1), 0)),
            pl.BlockSpec((1, 1, bj),
                         lambda g: (jnp.minimum(g, nblk - 1), 0, 0)),
            pl.BlockSpec((adim, bj),
                         lambda g: (0, jnp.clip(g - (nblk + 1), 0, nblk - 1))),
        ],
        out_specs=pl.BlockSpec((batch, adim), lambda g: (0, 0)),
        out_shape=jax.ShapeDtypeStruct((batch, adim), jnp.float32),
        scratch_shapes=[
            pltpu.VMEM((nblk, batch, bj), jnp.int32),
            pltpu.VMEM((nblk, 16, bj // 4), jnp.int32),
            pltpu.SMEM((2,), jnp.int32),
        ],
        compiler_params=pltpu.CompilerParams(
            dimension_semantics=("arbitrary",)),
        interpret=interpret,
    )(x_eff, W_enc, b_enc.reshape(nblk, 1, bj), W_dec)
    return y


def kernel(x, W_enc, b_enc, W_dec, b_dec):
    x_eff = x - b_dec[None, :]
    batch = x.shape[0]
    y = _run(x_eff, W_enc, b_enc, W_dec, k_total=64 * batch, bj=1024)
    return y + b_dec[None, :]


# speedup vs baseline: 7.2912x; 1.0272x over previous
import functools

import jax
import jax.numpy as jnp
from jax import lax
from jax.experimental import pallas as pl
from jax.experimental.pallas import tpu as pltpu


def _search_body(ai_ref, g12_ref, out_ref, *, nblk, batch, bj, fdim, k_total):
    def fold_add(m):
        s = m[0:8]
        for i in range(1, batch // 8):
            s = s + m[8 * i:8 * (i + 1)]
        t = s[:, 0:128]
        for i in range(1, bj // 128):
            t = t + s[:, 128 * i:128 * (i + 1)]
        return t

    def fold_with(m, op):
        s = m[0:8]
        for i in range(1, batch // 8):
            s = op(s, m[8 * i:8 * (i + 1)])
        t = s[:, 0:128]
        for i in range(1, bj // 128):
            t = op(t, s[:, 128 * i:128 * (i + 1)])
        return t

    def count_ge(t):
        def body(c, acc):
            blk = ai_ref[c]
            return acc + fold_add((blk >= t).astype(jnp.int32))
        acc = lax.fori_loop(0, nblk, body, jnp.zeros((8, 128), jnp.int32))
        return jnp.sum(acc)

    wc = bj // 4
    wf = min(wc, 128)

    unroll = 4 if nblk % 4 == 0 else 1

    def count_ge12(t):
        def one(c):
            blk = g12_ref[c]
            m = (blk >= t).astype(jnp.int32)
            s = m[0:8] + m[8:16]
            u = s[:, 0:wf]
            for i in range(1, wc // 128):
                u = u + s[:, 128 * i:128 * (i + 1)]
            return u
        def body(c, acc):
            for q in range(unroll):
                acc = acc + one(c * unroll + q)
            return acc
        acc = lax.fori_loop(0, nblk // unroll, body,
                            jnp.zeros((8, wf), jnp.int32))
        return jnp.sum(acc)

    def max12():
        def one(c):
            blk = g12_ref[c]
            s = jnp.maximum(blk[0:8], blk[8:16])
            u = s[:, 0:wf]
            for i in range(1, wc // 128):
                u = jnp.maximum(u, s[:, 128 * i:128 * (i + 1)])
            return u
        def body(c, acc):
            for q in range(unroll):
                acc = jnp.maximum(acc, one(c * unroll + q))
            return acc
        acc = lax.fori_loop(0, nblk // unroll, body,
                            jnp.zeros((8, wf), jnp.int32))
        return jnp.max(acc)

    def chunk_flat(c):
        return (lax.broadcasted_iota(jnp.int32, (batch, bj), 0) * fdim
                + lax.broadcasted_iota(jnp.int32, (batch, bj), 1) + c * bj)

    def count_tie_le(t, u):
        def body(c, acc):
            blk = ai_ref[c]
            m = (blk == t) & (chunk_flat(c) <= u)
            return acc + fold_add(m.astype(jnp.int32))
        acc = lax.fori_loop(0, nblk, body, jnp.zeros((8, 128), jnp.int32))
        return jnp.sum(acc)

    maxb = max12()

    def vcond(s):
        lo, hi = s
        return lo < hi

    def vbody12(s):
        lo, hi = s
        mid = lo + (hi - lo + 1) // 2
        big = count_ge12(mid) >= k_total
        return jnp.where(big, mid, lo), jnp.where(big, hi, mid - 1)

    tcand, _ = lax.while_loop(vcond, vbody12, (jnp.int32(0), maxb))

    ca = count_ge(tcand)
    cb = count_ge(tcand + 1)
    ok = (ca >= k_total) & (cb < k_total)

    def fallback():
        def vbody(s):
            lo, hi = s
            mid = lo + (hi - lo + 1) // 2
            big = count_ge(mid) >= k_total
            return jnp.where(big, mid, lo), jnp.where(big, hi, mid - 1)
        t_fb, _ = lax.while_loop(vcond, vbody, (jnp.int32(0), maxb))
        return t_fb, count_ge(t_fb + 1)

    tval, c_gt = lax.cond(ok, lambda: (tcand, cb), fallback)
    r = k_total - c_gt

    def min_tie():
        bigidx = jnp.int32(batch * fdim)
        def body(c, acc):
            blk = ai_ref[c]
            cand = jnp.where(blk == tval, chunk_flat(c), bigidx)
            return jnp.minimum(acc, fold_with(cand, jnp.minimum))
        acc = lax.fori_loop(
            0, nblk, body, jnp.full((8, 128), bigidx))
        return jnp.min(acc)

    def bisect_tie():
        def icond(s):
            lo, hi = s
            return lo < hi
        def ibody(s):
            lo, hi = s
            mid = lo + (hi - lo) // 2
            enough = count_tie_le(tval, mid) >= r
            return jnp.where(enough, lo, mid + 1), jnp.where(enough, mid, hi)
        lo, _ = lax.while_loop(
            icond, ibody, (jnp.int32(0), jnp.int32(batch * fdim - 1)))
        return lo

    u = lax.cond(r == 1, min_tie, bisect_tie)
    out_ref[0] = tval
    out_ref[1] = u


def _fused_body(x_ref, w_enc_ref, b_ref, w_dec_ref, out_ref, ai_scr, g12_scr,
                tpar, *, nblk, batch, bj, fdim, k_total):
    g = pl.program_id(0)

    @pl.when(g < nblk)
    def _encode():
        a = lax.dot_general(
            x_ref[...], w_enc_ref[...], (((1,), (1,)), ((), ())),
            preferred_element_type=jnp.float32)
        a = a + b_ref[0]
        a = jnp.maximum(a, 0.0)
        ai = lax.bitcast_convert_type(a, jnp.int32)
        ai = jnp.maximum(ai, 0)
        ai_scr[g] = ai
        nr = batch // 8
        if nr == 4:
            r0, r1 = ai[0:8], ai[8:16]
            r2, r3 = ai[16:24], ai[24:32]
            s1, i1 = jnp.maximum(r0, r1), jnp.minimum(r0, r1)
            s2, i2 = jnp.maximum(r2, r3), jnp.minimum(r2, r3)
            g1 = jnp.maximum(s1, s2)
            g2 = jnp.maximum(jnp.minimum(s1, s2), jnp.maximum(i1, i2))
        elif nr == 2:
            g1 = jnp.maximum(ai[0:8], ai[8:16])
            g2 = jnp.minimum(ai[0:8], ai[8:16])
        else:
            g1 = ai[0:8]
            g2 = jnp.zeros_like(g1)
        w = bj
        for _ in range(2):
            h = w // 2
            g1a, g1b = g1[:, :h], g1[:, h:]
            g2a, g2b = g2[:, :h], g2[:, h:]
            g1, g2 = (jnp.maximum(g1a, g1b),
                      jnp.maximum(jnp.minimum(g1a, g1b),
                                  jnp.maximum(g2a, g2b)))
            w = h
        g12_scr[g] = jnp.concatenate([g1, g2], axis=0)

    @pl.when(g == nblk)
    def _search():
        _search_body(ai_scr, g12_scr, tpar, nblk=nblk, batch=batch, bj=bj,
                     fdim=fdim, k_total=k_total)

    @pl.when(g > nblk)
    def _decode():
        j = g - (nblk + 1)
        tval = tpar[0]
        u = tpar[1]
        ai = ai_scr[j]
        flat = (lax.broadcasted_iota(jnp.int32, (batch, bj), 0) * fdim
                + lax.broadcasted_iota(jnp.int32, (batch, bj), 1) + j * bj)
        sel = (ai > tval) | ((ai == tval) & (flat <= u))
        vals = lax.bitcast_convert_type(ai, jnp.float32)
        f = jnp.where(sel, vals, 0.0)
        contrib = lax.dot_general(
            f, w_dec_ref[...], (((1,), (1,)), ((), ())),
            preferred_element_type=jnp.float32)

        @pl.when(j == 0)
        def _():
            out_ref[...] = contrib

        @pl.when(j > 0)
        def _():
            out_ref[...] += contrib


def _run(x_eff, W_enc, b_enc, W_dec, k_total, bj, interpret=False):
    batch, adim = x_eff.shape
    fdim = W_enc.shape[0]
    nblk = fdim // bj

    y = pl.pallas_call(
        functools.partial(_fused_body, nblk=nblk, batch=batch, bj=bj,
                          fdim=fdim, k_total=k_total),
        grid=(2 * nblk + 1,),
        in_specs=[
            pl.BlockSpec((batch, adim), lambda g: (0, 0)),
            pl.BlockSpec((bj, adim), lambda g: (jnp.minimum(g, nblk - 1), 0)),
            pl.BlockSpec((1, 1, bj),
                         lambda g: (jnp.minimum(g, nblk - 1), 0, 0)),
            pl.BlockSpec((adim, bj),
                         lambda g: (0, jnp.clip(g - (nblk + 1), 0, nblk - 1))),
        ],
        out_specs=pl.BlockSpec((batch, adim), lambda g: (0, 0)),
        out_shape=jax.ShapeDtypeStruct((batch, adim), jnp.float32),
        scratch_shapes=[
            pltpu.VMEM((nblk, batch, bj), jnp.int32),
            pltpu.VMEM((nblk, 16, bj // 4), jnp.int32),
            pltpu.SMEM((2,), jnp.int32),
        ],
        compiler_params=pltpu.CompilerParams(
            dimension_semantics=("arbitrary",)),
        interpret=interpret,
    )(x_eff, W_enc, b_enc.reshape(nblk, 1, bj), W_dec)
    return y


def kernel(x, W_enc, b_enc, W_dec, b_dec):
    x_eff = x - b_dec[None, :]
    batch = x.shape[0]
    y = _run(x_eff, W_enc, b_enc, W_dec, k_total=64 * batch, bj=1024)
    return y + b_dec[None, :]


# speedup vs baseline: 7.3022x; 1.0015x over previous
import functools

import jax
import jax.numpy as jnp
from jax import lax
from jax.experimental import pallas as pl
from jax.experimental.pallas import tpu as pltpu


def _search_body(ai_ref, g12_ref, out_ref, *, nblk, batch, bj, fdim, k_total):
    def fold_add(m):
        s = m[0:8]
        for i in range(1, batch // 8):
            s = s + m[8 * i:8 * (i + 1)]
        t = s[:, 0:128]
        for i in range(1, bj // 128):
            t = t + s[:, 128 * i:128 * (i + 1)]
        return t

    def fold_with(m, op):
        s = m[0:8]
        for i in range(1, batch // 8):
            s = op(s, m[8 * i:8 * (i + 1)])
        t = s[:, 0:128]
        for i in range(1, bj // 128):
            t = op(t, s[:, 128 * i:128 * (i + 1)])
        return t

    def count_ge(t):
        def body(c, acc):
            blk = ai_ref[c]
            return acc + fold_add((blk >= t).astype(jnp.int32))
        acc = lax.fori_loop(0, nblk, body, jnp.zeros((8, 128), jnp.int32))
        return jnp.sum(acc)

    def count_ge_pair(t):
        def body(c, accs):
            a0, a1 = accs
            blk = ai_ref[c]
            return (a0 + fold_add((blk >= t).astype(jnp.int32)),
                    a1 + fold_add((blk >= t + 1).astype(jnp.int32)))
        z = jnp.zeros((8, 128), jnp.int32)
        a0, a1 = lax.fori_loop(0, nblk, body, (z, z))
        return jnp.sum(a0), jnp.sum(a1)

    wc = bj // 4
    wf = min(wc, 128)

    unroll = 4 if nblk % 4 == 0 else 1

    def count_ge12(t):
        def one(c):
            blk = g12_ref[c]
            m = (blk >= t).astype(jnp.int32)
            s = m[0:8] + m[8:16]
            u = s[:, 0:wf]
            for i in range(1, wc // 128):
                u = u + s[:, 128 * i:128 * (i + 1)]
            return u
        def body(c, acc):
            for q in range(unroll):
                acc = acc + one(c * unroll + q)
            return acc
        acc = lax.fori_loop(0, nblk // unroll, body,
                            jnp.zeros((8, wf), jnp.int32))
        return jnp.sum(acc)

    def max12():
        def one(c):
            blk = g12_ref[c]
            s = jnp.maximum(blk[0:8], blk[8:16])
            u = s[:, 0:wf]
            for i in range(1, wc // 128):
                u = jnp.maximum(u, s[:, 128 * i:128 * (i + 1)])
            return u
        def body(c, acc):
            for q in range(unroll):
                acc = jnp.maximum(acc, one(c * unroll + q))
            return acc
        acc = lax.fori_loop(0, nblk // unroll, body,
                            jnp.zeros((8, wf), jnp.int32))
        return jnp.max(acc)

    def chunk_flat(c):
        return (lax.broadcasted_iota(jnp.int32, (batch, bj), 0) * fdim
                + lax.broadcasted_iota(jnp.int32, (batch, bj), 1) + c * bj)

    def count_tie_le(t, u):
        def body(c, acc):
            blk = ai_ref[c]
            m = (blk == t) & (chunk_flat(c) <= u)
            return acc + fold_add(m.astype(jnp.int32))
        acc = lax.fori_loop(0, nblk, body, jnp.zeros((8, 128), jnp.int32))
        return jnp.sum(acc)

    maxb = max12()

    def vcond(s):
        lo, hi = s
        return lo < hi

    def vbody12(s):
        lo, hi = s
        mid = lo + (hi - lo + 1) // 2
        big = count_ge12(mid) >= k_total
        return jnp.where(big, mid, lo), jnp.where(big, hi, mid - 1)

    tcand, _ = lax.while_loop(vcond, vbody12, (jnp.int32(0), maxb))

    ca, cb = count_ge_pair(tcand)
    ok = (ca >= k_total) & (cb < k_total)

    def fallback():
        def vbody(s):
            lo, hi = s
            mid = lo + (hi - lo + 1) // 2
            big = count_ge(mid) >= k_total
            return jnp.where(big, mid, lo), jnp.where(big, hi, mid - 1)
        t_fb, _ = lax.while_loop(vcond, vbody, (jnp.int32(0), maxb))
        return t_fb, count_ge(t_fb + 1)

    tval, c_gt = lax.cond(ok, lambda: (tcand, cb), fallback)
    r = k_total - c_gt

    def min_tie():
        bigidx = jnp.int32(batch * fdim)
        def body(c, acc):
            blk = ai_ref[c]
            cand = jnp.where(blk == tval, chunk_flat(c), bigidx)
            return jnp.minimum(acc, fold_with(cand, jnp.minimum))
        acc = lax.fori_loop(
            0, nblk, body, jnp.full((8, 128), bigidx))
        return jnp.min(acc)

    def bisect_tie():
        def icond(s):
            lo, hi = s
            return lo < hi
        def ibody(s):
            lo, hi = s
            mid = lo + (hi - lo) // 2
            enough = count_tie_le(tval, mid) >= r
            return jnp.where(enough, lo, mid + 1), jnp.where(enough, mid, hi)
        lo, _ = lax.while_loop(
            icond, ibody, (jnp.int32(0), jnp.int32(batch * fdim - 1)))
        return lo

    u = lax.cond(r == 1, min_tie, bisect_tie)
    out_ref[0] = tval
    out_ref[1] = u


def _fused_body(x_ref, w_enc_ref, b_ref, w_dec_ref, out_ref, ai_scr, g12_scr,
                tpar, *, nblk, batch, bj, fdim, k_total):
    g = pl.program_id(0)

    @pl.when(g < nblk)
    def _encode():
        a = lax.dot_general(
            x_ref[...], w_enc_ref[...], (((1,), (1,)), ((), ())),
            preferred_element_type=jnp.float32)
        a = a + b_ref[0]
        a = jnp.maximum(a, 0.0)
        ai = lax.bitcast_convert_type(a, jnp.int32)
        ai = jnp.maximum(ai, 0)
        ai_scr[g] = ai
        nr = batch // 8
        if nr == 4:
            r0, r1 = ai[0:8], ai[8:16]
            r2, r3 = ai[16:24], ai[24:32]
            s1, i1 = jnp.maximum(r0, r1), jnp.minimum(r0, r1)
            s2, i2 = jnp.maximum(r2, r3), jnp.minimum(r2, r3)
            g1 = jnp.maximum(s1, s2)
            g2 = jnp.maximum(jnp.minimum(s1, s2), jnp.maximum(i1, i2))
        elif nr == 2:
            g1 = jnp.maximum(ai[0:8], ai[8:16])
            g2 = jnp.minimum(ai[0:8], ai[8:16])
        else:
            g1 = ai[0:8]
            g2 = jnp.zeros_like(g1)
        w = bj
        for _ in range(2):
            h = w // 2
            g1a, g1b = g1[:, :h], g1[:, h:]
            g2a, g2b = g2[:, :h], g2[:, h:]
            g1, g2 = (jnp.maximum(g1a, g1b),
                      jnp.maximum(jnp.minimum(g1a, g1b),
                                  jnp.maximum(g2a, g2b)))
            w = h
        g12_scr[g] = jnp.concatenate([g1, g2], axis=0)

    @pl.when(g == nblk)
    def _search():
        _search_body(ai_scr, g12_scr, tpar, nblk=nblk, batch=batch, bj=bj,
                     fdim=fdim, k_total=k_total)

    @pl.when(g > nblk)
    def _decode():
        j = g - (nblk + 1)
        tval = tpar[0]
        u = tpar[1]
        ai = ai_scr[j]
        flat = (lax.broadcasted_iota(jnp.int32, (batch, bj), 0) * fdim
                + lax.broadcasted_iota(jnp.int32, (batch, bj), 1) + j * bj)
        sel = (ai > tval) | ((ai == tval) & (flat <= u))
        vals = lax.bitcast_convert_type(ai, jnp.float32)
        f = jnp.where(sel, vals, 0.0)
        contrib = lax.dot_general(
            f, w_dec_ref[...], (((1,), (1,)), ((), ())),
            preferred_element_type=jnp.float32)

        @pl.when(j == 0)
        def _():
            out_ref[...] = contrib

        @pl.when(j > 0)
        def _():
            out_ref[...] += contrib


def _run(x_eff, W_enc, b_enc, W_dec, k_total, bj, interpret=False):
    batch, adim = x_eff.shape
    fdim = W_enc.shape[0]
    nblk = fdim // bj

    y = pl.pallas_call(
        functools.partial(_fused_body, nblk=nblk, batch=batch, bj=bj,
                          fdim=fdim, k_total=k_total),
        grid=(2 * nblk + 1,),
        in_specs=[
            pl.BlockSpec((batch, adim), lambda g: (0, 0)),
            pl.BlockSpec((bj, adim), lambda g: (jnp.minimum(g, nblk - 1), 0)),
            pl.BlockSpec((1, 1, bj),
                         lambda g: (jnp.minimum(g, nblk - 1), 0, 0)),
            pl.BlockSpec((adim, bj),
                         lambda g: (0, jnp.clip(g - (nblk + 1), 0, nblk - 1))),
        ],
        out_specs=pl.BlockSpec((batch, adim), lambda g: (0, 0)),
        out_shape=jax.ShapeDtypeStruct((batch, adim), jnp.float32),
        scratch_shapes=[
            pltpu.VMEM((nblk, batch, bj), jnp.int32),
            pltpu.VMEM((nblk, 16, bj // 4), jnp.int32),
            pltpu.SMEM((2,), jnp.int32),
        ],
        compiler_params=pltpu.CompilerParams(
            dimension_semantics=("arbitrary",)),
        interpret=interpret,
    )(x_eff, W_enc, b_enc.reshape(nblk, 1, bj), W_dec)
    return y


def kernel(x, W_enc, b_enc, W_dec, b_dec):
    x_eff = x - b_dec[None, :]
    batch = x.shape[0]
    y = _run(x_eff, W_enc, b_enc, W_dec, k_total=64 * batch, bj=1024)
    return y + b_dec[None, :]
